# speedup vs baseline: 8.0992x; 8.0992x over previous
import functools

import jax
import jax.numpy as jnp
from jax import lax
from jax.experimental import pallas as pl
from jax.experimental.pallas import tpu as pltpu
from jax.experimental.pallas import tpu_sc as plsc

N_NODES = 10000
N_EDGES = 320000
D_FEAT = 128

NC = 2
NS = 16
NW = NC * NS
E_PER_W = N_EDGES // NW
EB = 80
NB = E_PER_W // EB
N_PAD = 10240
ROWS_PER_TILE = N_PAD // NS


def _sc_body(x_hbm, src_hbm, dst_hbm, zrows_hbm, zdeg_hbm, ones_hbm,
             acc_out, deg_out,
             src_v, dst_v, rows_v, ones_v, acc_sh, deg_sh, sem):
    cid = lax.axis_index("c")
    sid = lax.axis_index("s")
    wid = cid * NS + sid

    pltpu.sync_copy(src_hbm.at[wid], src_v)
    pltpu.sync_copy(dst_hbm.at[wid], dst_v)
    pltpu.sync_copy(ones_hbm, ones_v)

    stripe = pl.ds(sid * ROWS_PER_TILE, ROWS_PER_TILE)
    pltpu.sync_copy(zrows_hbm, acc_sh.at[stripe])
    pltpu.sync_copy(zdeg_hbm, deg_sh.at[stripe])
    plsc.subcore_barrier()

    def body(i, carry):
        pltpu.async_copy(x_hbm.at[src_v.at[i]], rows_v, sem).wait()
        pltpu.sync_copy(rows_v, acc_sh.at[dst_v.at[i]], add=True)
        pltpu.sync_copy(ones_v, deg_sh.at[dst_v.at[i]], add=True)
        return carry

    lax.fori_loop(0, NB, body, 0)
    plsc.subcore_barrier()

    pltpu.sync_copy(acc_sh.at[stripe], acc_out.at[cid, stripe])
    pltpu.sync_copy(deg_sh.at[stripe], deg_out.at[cid, stripe])


@functools.partial(
    pl.kernel,
    mesh=plsc.VectorSubcoreMesh(core_axis_name="c", subcore_axis_name="s"),
    out_type=[
        jax.ShapeDtypeStruct((NC, N_PAD, D_FEAT), jnp.float32),
        jax.ShapeDtypeStruct((NC, N_PAD), jnp.float32),
    ],
    scratch_types=[
        pltpu.VMEM((NB, EB), jnp.int32),
        pltpu.VMEM((NB, EB), jnp.int32),
        pltpu.VMEM((EB, D_FEAT), jnp.float32),
        pltpu.VMEM((EB,), jnp.float32),
        pltpu.VMEM_SHARED((N_PAD, D_FEAT), jnp.float32),
        pltpu.VMEM_SHARED((N_PAD,), jnp.float32),
        pltpu.SemaphoreType.DMA,
    ],
)
def _sc_aggregate(*refs):
    _sc_body(*refs)


_TC_R = 1000


def _tc_body(x_ref, p0_ref, p1_ref, d0_ref, d1_ref, ws_ref, wn_ref, b_ref,
             o_ref):
    deg = jnp.maximum(d0_ref[...] + d1_ref[...], 1.0)
    agg = (p0_ref[...] + p1_ref[...]) / deg
    h = jnp.dot(x_ref[...], ws_ref[...], preferred_element_type=jnp.float32)
    h = h + jnp.dot(agg, wn_ref[...], preferred_element_type=jnp.float32)
    o_ref[...] = jnp.maximum(h + b_ref[...], 0.0)


def _tc_finish(x, p0, p1, d0, d1, w_self, w_neigh, b2):
    grid = (N_NODES // _TC_R,)
    row_blk = pl.BlockSpec((_TC_R, D_FEAT), lambda i: (i, 0))
    col_blk = pl.BlockSpec((_TC_R, 1), lambda i: (i, 0))
    full_w = pl.BlockSpec((D_FEAT, D_FEAT), lambda i: (0, 0))
    return pl.pallas_call(
        _tc_body,
        grid=grid,
        in_specs=[row_blk, row_blk, row_blk, col_blk, col_blk, full_w,
                  full_w, pl.BlockSpec((1, D_FEAT), lambda i: (0, 0))],
        out_specs=row_blk,
        out_shape=jax.ShapeDtypeStruct((N_NODES, D_FEAT), jnp.float32),
    )(x, p0, p1, d0, d1, w_self, w_neigh, b2)


def kernel(x, edge_index, batch, W_self, W_neigh, b):
    src = edge_index[0].astype(jnp.int32).reshape(NW, NB, EB)
    dst = edge_index[1].astype(jnp.int32).reshape(NW, NB, EB)
    zrows = jnp.zeros((ROWS_PER_TILE, D_FEAT), jnp.float32)
    zdeg = jnp.zeros((ROWS_PER_TILE,), jnp.float32)
    ones = jnp.ones((EB,), jnp.float32)

    acc, deg = _sc_aggregate(x, src, dst, zrows, zdeg, ones)

    p0 = acc[0, :N_NODES]
    p1 = acc[1, :N_NODES]
    d0 = deg[0, :N_NODES].reshape(N_NODES, 1)
    d1 = deg[1, :N_NODES].reshape(N_NODES, 1)
    out = _tc_finish(x, p0, p1, d0, d1, W_self, W_neigh,
                     b.reshape(1, D_FEAT))
    return out, batch


# speedup vs baseline: 9.6169x; 1.1874x over previous
import functools

import jax
import jax.numpy as jnp
from jax import lax
from jax.experimental import pallas as pl
from jax.experimental.pallas import tpu as pltpu
from jax.experimental.pallas import tpu_sc as plsc

N_NODES = 10000
N_EDGES = 320000
D_FEAT = 128

NC = 2
NS = 16
NW = NC * NS
E_PER_W = N_EDGES // NW
EB = 80
NB = E_PER_W // EB
N_PAD = 10240
ROWS_PER_TILE = N_PAD // NS
Z_ROWS = 128


def _sc_body(x_hbm, pk_hbm, zrows_hbm, zdeg_hbm, ones_hbm,
             acc_out, deg_out,
             pk_v, sb0, db0, sb1, db1, buf0, buf1, ones_v, acc_sh, deg_sh,
             sem0, sem1, semd):
    cid = lax.axis_index("c")
    sid = lax.axis_index("s")
    wid = cid * NS + sid

    pltpu.sync_copy(pk_hbm.at[wid], pk_v)
    pltpu.sync_copy(ones_hbm, ones_v)

    stripe = pl.ds(sid * ROWS_PER_TILE, ROWS_PER_TILE)
    for z in range(ROWS_PER_TILE // Z_ROWS):
        pltpu.sync_copy(
            zrows_hbm, acc_sh.at[pl.ds(sid * ROWS_PER_TILE + z * Z_ROWS,
                                       Z_ROWS)])
    pltpu.sync_copy(zdeg_hbm, deg_sh.at[stripe])
    plsc.subcore_barrier()

    def unpack(i, src_b, dst_b):
        for k in range(EB // 16):
            w = pk_v[i, pl.ds(k * 16, 16)]
            src_b[pl.ds(k * 16, 16)] = w & 0xFFFF
            dst_b[pl.ds(k * 16, 16)] = lax.shift_right_logical(w, 16)

    def start_gather(src_b, buf, sem):
        pltpu.async_copy(x_hbm.at[src_b], buf, sem)

    def wait_gather(src_b, buf, sem):
        pltpu.make_async_copy(x_hbm.at[src_b], buf, sem).wait()

    unpack(0, sb0, db0)
    start_gather(sb0, buf0, sem0)
    unpack(1, sb1, db1)
    start_gather(sb1, buf1, sem1)

    def body(g, carry):
        i0 = 2 * g
        wait_gather(sb0, buf0, sem0)
        pltpu.sync_copy(buf0, acc_sh.at[db0], add=True)
        pltpu.async_copy(ones_v, deg_sh.at[db0], semd, add=True)
        wait_gather(sb1, buf1, sem1)
        pltpu.sync_copy(buf1, acc_sh.at[db1], add=True)
        pltpu.async_copy(ones_v, deg_sh.at[db1], semd, add=True)
        pltpu.make_async_copy(ones_v, deg_sh.at[db0], semd).wait()
        pltpu.make_async_copy(ones_v, deg_sh.at[db1], semd).wait()
        unpack(i0 + 2, sb0, db0)
        start_gather(sb0, buf0, sem0)
        unpack(i0 + 3, sb1, db1)
        start_gather(sb1, buf1, sem1)
        return carry

    lax.fori_loop(0, (NB - 3) // 2, body, 0)

    wait_gather(sb0, buf0, sem0)
    pltpu.sync_copy(buf0, acc_sh.at[db0], add=True)
    pltpu.async_copy(ones_v, deg_sh.at[db0], semd, add=True)
    wait_gather(sb1, buf1, sem1)
    pltpu.sync_copy(buf1, acc_sh.at[db1], add=True)
    pltpu.async_copy(ones_v, deg_sh.at[db1], semd, add=True)
    pltpu.make_async_copy(ones_v, deg_sh.at[db0], semd).wait()
    pltpu.make_async_copy(ones_v, deg_sh.at[db1], semd).wait()
    unpack(NB - 1, sb0, db0)
    start_gather(sb0, buf0, sem0)
    wait_gather(sb0, buf0, sem0)
    pltpu.sync_copy(buf0, acc_sh.at[db0], add=True)
    pltpu.sync_copy(ones_v, deg_sh.at[db0], add=True)
    plsc.subcore_barrier()

    pltpu.sync_copy(acc_sh.at[stripe], acc_out.at[cid, stripe])
    pltpu.sync_copy(deg_sh.at[stripe], deg_out.at[cid, stripe])


@functools.partial(
    pl.kernel,
    mesh=plsc.VectorSubcoreMesh(core_axis_name="c", subcore_axis_name="s"),
    out_type=[
        jax.ShapeDtypeStruct((NC, N_PAD, D_FEAT), jnp.float32),
        jax.ShapeDtypeStruct((NC, N_PAD), jnp.float32),
    ],
    scratch_types=[
        pltpu.VMEM((NB, EB), jnp.int32),
        pltpu.VMEM((EB,), jnp.int32),
        pltpu.VMEM((EB,), jnp.int32),
        pltpu.VMEM((EB,), jnp.int32),
        pltpu.VMEM((EB,), jnp.int32),
        pltpu.VMEM((EB, D_FEAT), jnp.float32),
        pltpu.VMEM((EB, D_FEAT), jnp.float32),
        pltpu.VMEM((EB,), jnp.float32),
        pltpu.VMEM_SHARED((N_PAD, D_FEAT), jnp.float32),
        pltpu.VMEM_SHARED((N_PAD,), jnp.float32),
        pltpu.SemaphoreType.DMA,
        pltpu.SemaphoreType.DMA,
        pltpu.SemaphoreType.DMA,
    ],
)
def _sc_aggregate(*refs):
    _sc_body(*refs)


_TC_R = 1000


def _tc_body(x_ref, p0_ref, p1_ref, d0_ref, d1_ref, ws_ref, wn_ref, b_ref,
             o_ref):
    deg = jnp.maximum(d0_ref[...] + d1_ref[...], 1.0)
    agg = (p0_ref[...] + p1_ref[...]) / deg
    h = jnp.dot(x_ref[...], ws_ref[...], preferred_element_type=jnp.float32)
    h = h + jnp.dot(agg, wn_ref[...], preferred_element_type=jnp.float32)
    o_ref[...] = jnp.maximum(h + b_ref[...], 0.0)


def _tc_finish(x, p0, p1, d0, d1, w_self, w_neigh, b2):
    grid = (N_NODES // _TC_R,)
    row_blk = pl.BlockSpec((_TC_R, D_FEAT), lambda i: (i, 0))
    col_blk = pl.BlockSpec((_TC_R, 1), lambda i: (i, 0))
    full_w = pl.BlockSpec((D_FEAT, D_FEAT), lambda i: (0, 0))
    return pl.pallas_call(
        _tc_body,
        grid=grid,
        in_specs=[row_blk, row_blk, row_blk, col_blk, col_blk, full_w,
                  full_w, pl.BlockSpec((1, D_FEAT), lambda i: (0, 0))],
        out_specs=row_blk,
        out_shape=jax.ShapeDtypeStruct((N_NODES, D_FEAT), jnp.float32),
    )(x, p0, p1, d0, d1, w_self, w_neigh, b2)


def kernel(x, edge_index, batch, W_self, W_neigh, b):
    src = edge_index[0].astype(jnp.int32)
    dst = edge_index[1].astype(jnp.int32)
    packed = (src | (dst << 16)).reshape(NW, NB, EB)
    zrows = jnp.zeros((Z_ROWS, D_FEAT), jnp.float32)
    zdeg = jnp.zeros((ROWS_PER_TILE,), jnp.float32)
    ones = jnp.ones((EB,), jnp.float32)

    acc, deg = _sc_aggregate(x, packed, zrows, zdeg, ones)

    p0 = acc[0, :N_NODES]
    p1 = acc[1, :N_NODES]
    d0 = deg[0, :N_NODES].reshape(N_NODES, 1)
    d1 = deg[1, :N_NODES].reshape(N_NODES, 1)
    out = _tc_finish(x, p0, p1, d0, d1, W_self, W_neigh,
                     b.reshape(1, D_FEAT))
    return out, batch
